# baseline (speedup 1.0000x reference)
import jax
import jax.numpy as jnp
from jax.experimental import pallas as pl
from jax.experimental.pallas import tpu as pltpu

_LANES = 128


def _make_body(group, groups_per_block):
    def _blend_body(x_sref, a_ref, y_ref, e_ref, o_ref):
        eb = e_ref[...]
        yv = y_ref[...]
        rb = pl.program_id(0)
        for k in range(groups_per_block):
            sl = pl.ds(k * group, group)
            a = a_ref[sl, :]
            m = x_sref[rb * groups_per_block + k] * yv
            o_ref[sl, :] = a + m * (eb - a)
    return _blend_body


def kernel(arr, element, x_index, y_index):
    M, N, D = arr.shape
    R = M * N * D // _LANES
    group = N * D // _LANES
    gpb = min(32, M)
    rows = group * gpb
    a2 = arr.reshape(R, _LANES)
    y2 = jnp.repeat(y_index, D).reshape(group, _LANES)
    e2 = jnp.tile(element, _LANES // D).reshape(1, _LANES)

    grid_spec = pltpu.PrefetchScalarGridSpec(
        num_scalar_prefetch=1,
        grid=(R // rows,),
        in_specs=[
            pl.BlockSpec((rows, _LANES), lambda i, xs: (i, 0)),
            pl.BlockSpec((group, _LANES), lambda i, xs: (0, 0)),
            pl.BlockSpec((1, _LANES), lambda i, xs: (0, 0)),
        ],
        out_specs=pl.BlockSpec((rows, _LANES), lambda i, xs: (i, 0)),
    )
    out = pl.pallas_call(
        _make_body(group, gpb),
        grid_spec=grid_spec,
        out_shape=jax.ShapeDtypeStruct((R, _LANES), jnp.float32),
    )(x_index, a2, y2, e2)
    return out.reshape(M, N, D)


# speedup vs baseline: 36.1868x; 36.1868x over previous
import jax
import jax.numpy as jnp
from jax.experimental import pallas as pl


def _blend_body(a_ref, x_ref, y_ref, e_ref, o_ref):
    a = a_ref[...]
    m = x_ref[...] * y_ref[...]
    o_ref[...] = a + m * (e_ref[...] - a)


def kernel(arr, element, x_index, y_index):
    M, N, D = arr.shape
    at = arr.transpose(0, 2, 1)
    x3 = x_index.reshape(M, 1, 1)
    y3 = y_index.reshape(1, 1, N)
    e3 = element.reshape(1, D, 1)

    BM = min(32, M)
    BN = min(4096, N)
    grid = (M // BM, N // BN)

    out = pl.pallas_call(
        _blend_body,
        grid=grid,
        in_specs=[
            pl.BlockSpec((BM, D, BN), lambda i, j: (i, 0, j)),
            pl.BlockSpec((BM, 1, 1), lambda i, j: (i, 0, 0)),
            pl.BlockSpec((1, 1, BN), lambda i, j: (0, 0, j)),
            pl.BlockSpec((1, D, 1), lambda i, j: (0, 0, 0)),
        ],
        out_specs=pl.BlockSpec((BM, D, BN), lambda i, j: (i, 0, j)),
        out_shape=jax.ShapeDtypeStruct((M, D, N), jnp.float32),
    )(at, x3, y3, e3)
    return out.transpose(0, 2, 1)
